# Initial kernel scaffold; baseline (speedup 1.0000x reference)
import jax
import jax.numpy as jnp
from jax.experimental import pallas as pl


def kernel(pos, batch, W1a, b1a, W1b, b1b, W2, b2, W3, b3, W4a, b4a, W4b, b4b, W4c, b4c):
    raise NotImplementedError("write your pallas kernel here")


# speedup vs baseline: 5.4242x; 5.4242x over previous
import functools

import jax
import jax.numpy as jnp
from jax import lax
from jax.experimental import pallas as pl
from jax.experimental.pallas import tpu as pltpu
from jax.experimental.pallas import tpu_sc as plsc

_N = 8192
_K = 20
_G = 8
_NEG = -3.4e38


def _knn_body(x_ref, xt_ref, bcol_ref, brow_ref, out_ref, d_s):
    xr = x_ref[...]
    xt = xt_ref[...]
    sqr = jnp.sum(xr * xr, axis=1, keepdims=True)
    sqc = jnp.sum(xt * xt, axis=0, keepdims=True)
    xy = lax.dot_general(xr, xt, (((1,), (0,)), ((), ())),
                         preferred_element_type=jnp.float32)
    d = sqr + sqc - 2.0 * xy
    bm = bcol_ref[:, 0:1] != brow_ref[...]
    d = jnp.where(bm, jnp.float32(1e10), d)
    d_s[...] = d

    r_dim, n = d.shape
    lane = lax.broadcasted_iota(jnp.int32, (r_dim, n), 1)
    col32 = lax.broadcasted_iota(jnp.int32, (r_dim, 32), 1)

    def body(r, acc):
        dcur = d_s[...]
        m = jnp.min(dcur, axis=1, keepdims=True)
        idx = jnp.min(jnp.where(dcur == m, lane, jnp.int32(2**30)),
                      axis=1, keepdims=True)
        d_s[...] = jnp.where(lane == idx, jnp.float32(3e38), dcur)
        return jnp.where(col32 == r, idx, acc)

    out_ref[...] = lax.fori_loop(0, _K, body,
                                 jnp.zeros((r_dim, 32), jnp.int32))


def _knn(x, xt, bcol, brow, rows=128):
    n, f = x.shape
    return pl.pallas_call(
        _knn_body,
        grid=(n // rows,),
        in_specs=[
            pl.BlockSpec((rows, f), lambda i: (i, 0)),
            pl.BlockSpec((f, n), lambda i: (0, 0)),
            pl.BlockSpec((rows, 128), lambda i: (i, 0)),
            pl.BlockSpec((1, n), lambda i: (0, 0)),
        ],
        out_specs=pl.BlockSpec((rows, 32), lambda i: (i, 0)),
        out_shape=jax.ShapeDtypeStruct((n, 32), jnp.int32),
        scratch_shapes=[pltpu.VMEM((rows, n), jnp.float32)],
    )(x, xt, bcol, brow)


def _sc_gather(table, idx3):
    v_dim, d_dim = table.shape
    nw, n_dma, _ = idx3.shape
    gsz = 4
    ngrp = n_dma // gsz
    mesh = plsc.VectorSubcoreMesh(core_axis_name="c", subcore_axis_name="s")

    @functools.partial(
        pl.kernel,
        mesh=mesh,
        out_type=jax.ShapeDtypeStruct((nw, n_dma, 128, d_dim), jnp.float32),
        scratch_types=[
            pltpu.VMEM((n_dma, 128), jnp.int32),
            pltpu.VMEM((gsz, 128, d_dim), jnp.float32),
            pltpu.SemaphoreType.DMA,
        ],
    )
    def gk(table_hbm, idx_hbm, out_hbm, idx_v, rows_v, sem):
        wid = lax.axis_index("s") * 2 + lax.axis_index("c")
        pltpu.sync_copy(idx_hbm.at[wid], idx_v)

        def grp(g, carry):
            handles = []
            for b in range(gsz):
                handles.append(pltpu.async_copy(
                    table_hbm.at[idx_v.at[g * gsz + b]], rows_v.at[b], sem))
            for h in handles:
                h.wait()
            pltpu.sync_copy(rows_v, out_hbm.at[wid, pl.ds(g * gsz, gsz)])
            return carry

        lax.fori_loop(0, ngrp, grp, 0)

    return gk(table, idx3)


def _lin_body(x_ref, w_ref, b_ref, out_ref):
    out_ref[...] = lax.dot_general(
        x_ref[...], w_ref[...], (((1,), (0,)), ((), ())),
        preferred_element_type=jnp.float32) + b_ref[...]


def _lin(x, w, b, rows=1024):
    n, f = x.shape
    dout = w.shape[1]
    return pl.pallas_call(
        _lin_body,
        grid=(n // rows,),
        in_specs=[
            pl.BlockSpec((rows, f), lambda i: (i, 0)),
            pl.BlockSpec((f, dout), lambda i: (0, 0)),
            pl.BlockSpec((1, dout), lambda i: (0, 0)),
        ],
        out_specs=pl.BlockSpec((rows, dout), lambda i: (i, 0)),
        out_shape=jax.ShapeDtypeStruct((n, dout), jnp.float32),
    )(x, w, b)


def _conv1_body(x_ref, g_ref, w1i_ref, b1a_ref, w1b_ref, b1b_ref, out_ref):
    dn = (((1,), (0,)), ((), ()))
    a = lax.dot_general(x_ref[...], w1i_ref[...], dn,
                        preferred_element_type=jnp.float32) + b1a_ref[...]
    acc = jnp.zeros(out_ref.shape, jnp.float32)
    for k in range(_K):
        h = jnp.maximum(a + g_ref[k][:, :64], 0.0)
        h2 = lax.dot_general(h, w1b_ref[...], dn,
                             preferred_element_type=jnp.float32) + b1b_ref[...]
        acc = jnp.maximum(acc, jnp.maximum(h2, 0.0))
    out_ref[...] = acc


def _conv1(x16, g3, w1i, b1a, w1b, b1b, rows=512):
    n = x16.shape[0]
    return pl.pallas_call(
        _conv1_body,
        grid=(n // rows,),
        in_specs=[
            pl.BlockSpec((rows, 16), lambda i: (i, 0)),
            pl.BlockSpec((_K, rows, 128), lambda i: (0, i, 0)),
            pl.BlockSpec((16, 64), lambda i: (0, 0)),
            pl.BlockSpec((1, 64), lambda i: (0, 0)),
            pl.BlockSpec((64, 64), lambda i: (0, 0)),
            pl.BlockSpec((1, 64), lambda i: (0, 0)),
        ],
        out_specs=pl.BlockSpec((rows, 64), lambda i: (i, 0)),
        out_shape=jax.ShapeDtypeStruct((n, 64), jnp.float32),
    )(x16, g3, w1i, b1a, w1b, b1b)


def _conv2_body(a_ref, g_ref, out_ref):
    a = a_ref[...]
    acc = jnp.zeros(out_ref.shape, jnp.float32)
    for k in range(_K):
        acc = jnp.maximum(acc, jnp.maximum(a + g_ref[k], 0.0))
    out_ref[...] = acc


def _conv2(a2, g3, rows=512):
    n = a2.shape[0]
    return pl.pallas_call(
        _conv2_body,
        grid=(n // rows,),
        in_specs=[
            pl.BlockSpec((rows, 128), lambda i: (i, 0)),
            pl.BlockSpec((_K, rows, 128), lambda i: (0, i, 0)),
        ],
        out_specs=pl.BlockSpec((rows, 128), lambda i: (i, 0)),
        out_shape=jax.ShapeDtypeStruct((n, 128), jnp.float32),
    )(a2, g3)


def _final_body(o1_ref, o2_ref, bcol_ref, w3a_ref, w3b_ref, b3_ref,
                w4at_ref, w4ab_ref, b4a_ref, w4b_ref, b4b_ref,
                w4c_ref, b4c_ref, out_ref, ssum, smax, scnt):
    dn = (((1,), (0,)), ((), ()))
    dnt = (((0,), (0,)), ((), ()))
    i = pl.program_id(0)

    @pl.when(i == 0)
    def _():
        ssum[...] = jnp.zeros_like(ssum)
        smax[...] = jnp.full_like(smax, _NEG)
        scnt[...] = jnp.zeros_like(scnt)

    y = lax.dot_general(o1_ref[...], w3a_ref[...], dn,
                        preferred_element_type=jnp.float32)
    y = y + lax.dot_general(o2_ref[...], w3b_ref[...], dn,
                            preferred_element_type=jnp.float32)
    y = jnp.maximum(y + b3_ref[...], 0.0)

    br = bcol_ref[:, 0:1]
    r_dim = y.shape[0]
    gio = lax.broadcasted_iota(jnp.int32, (r_dim, _G), 1)
    oh = (br == gio).astype(jnp.float32)
    ssum[...] = ssum[...] + lax.dot_general(
        oh, y, dnt, preferred_element_type=jnp.float32)
    ones = jnp.ones((r_dim, 128), jnp.float32)
    scnt[...] = scnt[...] + lax.dot_general(
        oh, ones, dnt, preferred_element_type=jnp.float32)
    for g in range(_G):
        mg = jnp.max(jnp.where(br == g, y, _NEG), axis=0, keepdims=True)
        smax[g:g + 1, :] = jnp.maximum(smax[g:g + 1, :], mg)

    @pl.when(i == pl.num_programs(0) - 1)
    def _():
        cnt = jnp.maximum(scnt[:, 0:1], 1.0)
        mean = ssum[...] / cnt
        h = lax.dot_general(mean, w4at_ref[...], dn,
                            preferred_element_type=jnp.float32)
        h = h + lax.dot_general(smax[...], w4ab_ref[...], dn,
                                preferred_element_type=jnp.float32)
        h = jnp.maximum(h + b4a_ref[...], 0.0)
        h = jnp.maximum(
            lax.dot_general(h, w4b_ref[...], dn,
                            preferred_element_type=jnp.float32)
            + b4b_ref[...], 0.0)
        out_ref[...] = lax.dot_general(
            h, w4c_ref[...], dn,
            preferred_element_type=jnp.float32) + b4c_ref[...]


def _final(o1, o2, bcol, w3a, w3b, b3, w4at, w4ab, b4a, w4b, b4b, w4c, b4c,
           rows=512):
    n = o1.shape[0]
    return pl.pallas_call(
        _final_body,
        grid=(n // rows,),
        in_specs=[
            pl.BlockSpec((rows, 64), lambda i: (i, 0)),
            pl.BlockSpec((rows, 128), lambda i: (i, 0)),
            pl.BlockSpec((rows, 128), lambda i: (i, 0)),
            pl.BlockSpec((64, 512), lambda i: (0, 0)),
            pl.BlockSpec((128, 512), lambda i: (0, 0)),
            pl.BlockSpec((1, 512), lambda i: (0, 0)),
            pl.BlockSpec((512, 512), lambda i: (0, 0)),
            pl.BlockSpec((512, 512), lambda i: (0, 0)),
            pl.BlockSpec((1, 512), lambda i: (0, 0)),
            pl.BlockSpec((512, 256), lambda i: (0, 0)),
            pl.BlockSpec((1, 256), lambda i: (0, 0)),
            pl.BlockSpec((256, 10), lambda i: (0, 0)),
            pl.BlockSpec((1, 10), lambda i: (0, 0)),
        ],
        out_specs=pl.BlockSpec((_G, 10), lambda i: (0, 0)),
        out_shape=jax.ShapeDtypeStruct((_G, 10), jnp.float32),
        scratch_shapes=[
            pltpu.VMEM((_G, 512), jnp.float32),
            pltpu.VMEM((_G, 512), jnp.float32),
            pltpu.VMEM((_G, 128), jnp.float32),
        ],
    )(o1, o2, bcol, w3a, w3b, b3, w4at, w4ab, b4a, w4b, b4b, w4c, b4c)


def kernel(pos, batch, W1a, b1a, W1b, b1b, W2, b2, W3, b3, W4a, b4a,
           W4b, b4b, W4c, b4c):
    posp = jnp.pad(pos, ((0, 0), (0, 13)))
    bcol = jnp.broadcast_to(batch[:, None], (_N, 128))
    brow = batch.reshape(1, _N)

    idx1 = _knn(posp, posp.T, bcol, brow)[:, :_K]
    w1i = jnp.pad(W1a[:3] - W1a[3:], ((0, 13), (0, 0)))
    w1j128 = jnp.pad(W1a[3:], ((0, 13), (0, 64)))
    c1 = _lin(posp, w1j128, jnp.zeros((1, 128), jnp.float32))
    g1 = _sc_gather(c1, idx1.T.reshape(32, -1, 128))
    g1 = g1.reshape(_K, _N, 128)
    out1 = _conv1(posp, g1, w1i, b1a.reshape(1, -1), W1b,
                  b1b.reshape(1, -1))

    idx2 = _knn(out1, out1.T, bcol, brow)[:, :_K]
    a2 = _lin(out1, W2[:64] - W2[64:], b2.reshape(1, -1))
    c2 = _lin(out1, W2[64:], jnp.zeros((1, 128), jnp.float32))
    g2 = _sc_gather(c2, idx2.T.reshape(32, -1, 128))
    g2 = g2.reshape(_K, _N, 128)
    out2 = _conv2(a2, g2)

    return _final(out1, out2, bcol, W3[:64], W3[64:], b3.reshape(1, -1),
                  W4a[:512], W4a[512:], b4a.reshape(1, -1),
                  W4b, b4b.reshape(1, -1), W4c, b4c.reshape(1, -1))
